# Initial kernel scaffold; baseline (speedup 1.0000x reference)
import functools

import jax
import jax.numpy as jnp
from jax import lax
from jax.experimental import pallas as pl
from jax.experimental.pallas import tpu as pltpu
from jax.experimental.pallas import tpu_sc as plsc

_N = 10000
_D = 128
_E = 320000
_NC = 2
_NS = 16
_EPW = _E // (_NC * _NS)
_CH = 128
_NFULL = _EPW // _CH
_REM = _EPW - _NFULL * _CH
_RPT = _N // _NS
_ZCH = 125
_NLANE = _D // 16


@functools.partial(
    pl.kernel,
    out_type=jax.ShapeDtypeStruct((_NC, _N, _D), jnp.float32),
    mesh=plsc.VectorSubcoreMesh(core_axis_name="c", subcore_axis_name="s"),
    scratch_types=[
        pltpu.VMEM((_CH,), jnp.int32),
        pltpu.VMEM((_CH,), jnp.int32),
        pltpu.VMEM((_CH,), jnp.float32),
        pltpu.VMEM((_CH, _D), jnp.float32),
        pltpu.VMEM((_REM,), jnp.int32),
        pltpu.VMEM((_REM,), jnp.int32),
        pltpu.VMEM((_REM,), jnp.float32),
        pltpu.VMEM((_REM, _D), jnp.float32),
        pltpu.VMEM_SHARED((_N, _D), jnp.float32),
        pltpu.SemaphoreType.DMA,
    ],
)
def _sc_aggregate(x_hbm, src_hbm, dst_hbm, w_hbm, out_hbm,
                  src_v, dst_v, w_v, rows_v, src_r, dst_r, w_r, rows_r,
                  acc, sem):
    c = lax.axis_index("c")
    s = lax.axis_index("s")

    zero16 = jnp.zeros((16,), jnp.float32)

    def _zero_row(i, carry):
        for k in range(_NLANE):
            rows_v[i, pl.ds(k * 16, 16)] = zero16
        return carry

    lax.fori_loop(0, _ZCH, _zero_row, 0)
    for j in range(_RPT // _ZCH):
        pltpu.sync_copy(rows_v.at[pl.ds(0, _ZCH)],
                        acc.at[pl.ds(s * _RPT + j * _ZCH, _ZCH)])
    plsc.subcore_barrier()

    base = (c * _NS + s) * _EPW

    def _scale_rows(rows_ref, wts_ref, count):
        def _one_edge(e, carry):
            wb = plsc.load_gather(wts_ref, [jnp.full((16,), e, jnp.int32)])
            for k in range(_NLANE):
                rows_ref[e, pl.ds(k * 16, 16)] = (
                    rows_ref[e, pl.ds(k * 16, 16)] * wb)
            return carry
        lax.fori_loop(0, count, _one_edge, 0)

    def _chunk(i, carry):
        off = base + i * _CH
        pltpu.sync_copy(src_hbm.at[pl.ds(off, _CH)], src_v)
        pltpu.sync_copy(dst_hbm.at[pl.ds(off, _CH)], dst_v)
        pltpu.sync_copy(w_hbm.at[pl.ds(off, _CH)], w_v)
        pltpu.async_copy(x_hbm.at[src_v], rows_v, sem).wait()
        _scale_rows(rows_v, w_v, _CH)
        pltpu.sync_copy(rows_v, acc.at[dst_v], add=True)
        return carry

    lax.fori_loop(0, _NFULL, _chunk, 0)

    if _REM:
        off = base + _NFULL * _CH
        pltpu.sync_copy(src_hbm.at[pl.ds(off, _REM)], src_r)
        pltpu.sync_copy(dst_hbm.at[pl.ds(off, _REM)], dst_r)
        pltpu.sync_copy(w_hbm.at[pl.ds(off, _REM)], w_r)
        pltpu.async_copy(x_hbm.at[src_r], rows_r, sem).wait()
        _scale_rows(rows_r, w_r, _REM)
        pltpu.sync_copy(rows_r, acc.at[dst_r], add=True)

    plsc.subcore_barrier()
    pltpu.sync_copy(acc.at[pl.ds(s * _RPT, _RPT)],
                    out_hbm.at[c, pl.ds(s * _RPT, _RPT)])


_BM = 1000


def _tc_body(p_ref, w_ref, o_ref):
    agg = p_ref[0] + p_ref[1]
    o_ref[...] = jnp.maximum(
        jnp.dot(agg, w_ref[...], preferred_element_type=jnp.float32), 0.0)


def _tc_matmul_relu(partials, W):
    return pl.pallas_call(
        _tc_body,
        grid=(_N // _BM,),
        in_specs=[
            pl.BlockSpec((_NC, _BM, _D), lambda i: (0, i, 0)),
            pl.BlockSpec((_D, _D), lambda i: (0, 0)),
        ],
        out_specs=pl.BlockSpec((_BM, _D), lambda i: (i, 0)),
        out_shape=jax.ShapeDtypeStruct((_N, _D), jnp.float32),
    )(partials, W)


@jax.jit
def kernel(x, edge_index, edge_weight, W):
    src = edge_index[0].astype(jnp.int32)
    dst = edge_index[1].astype(jnp.int32)
    partials = _sc_aggregate(x, src, dst, edge_weight.astype(jnp.float32))
    return _tc_matmul_relu(partials, W)


# speedup vs baseline: 5.1300x; 5.1300x over previous
import functools

import jax
import jax.numpy as jnp
from jax import lax
from jax.experimental import pallas as pl
from jax.experimental.pallas import tpu as pltpu
from jax.experimental.pallas import tpu_sc as plsc

_N = 10000
_D = 128
_E = 320000
_NC = 2
_NS = 16
_EPW = _E // (_NC * _NS)
_CH = 128
_NFULL = _EPW // _CH
_REM = _EPW - _NFULL * _CH
_RSPAN = 624
_TAIL = _N - _NS * _RSPAN
_ZSIZES = (128, 128, 128, 128, 112)
_NLANE = _D // 16


@functools.partial(
    pl.kernel,
    out_type=jax.ShapeDtypeStruct((_NC, _N, _D), jnp.float32),
    mesh=plsc.VectorSubcoreMesh(core_axis_name="c", subcore_axis_name="s"),
    scratch_types=[
        pltpu.VMEM((_CH,), jnp.int32),
        pltpu.VMEM((_CH,), jnp.int32),
        pltpu.VMEM((_CH,), jnp.float32),
        pltpu.VMEM((_CH, _D), jnp.float32),
        pltpu.VMEM((_REM,), jnp.int32),
        pltpu.VMEM((_REM,), jnp.int32),
        pltpu.VMEM((_REM,), jnp.float32),
        pltpu.VMEM((_REM, _D), jnp.float32),
        pltpu.VMEM_SHARED((_N, _D), jnp.float32),
        pltpu.SemaphoreType.DMA,
    ],
)
def _sc_aggregate(x_hbm, src_hbm, dst_hbm, w_hbm, out_hbm,
                  src_v, dst_v, w_v, rows_v, src_r, dst_r, w_r, rows_r,
                  acc, sem):
    c = lax.axis_index("c")
    s = lax.axis_index("s")

    zero16 = jnp.zeros((16,), jnp.float32)

    def _zero_row(i, carry):
        for k in range(_NLANE):
            rows_v[i, pl.ds(k * 16, 16)] = zero16
        return carry

    lax.fori_loop(0, _CH, _zero_row, 0)
    zoff = 0
    for zsz in _ZSIZES:
        pltpu.sync_copy(rows_v.at[pl.ds(0, zsz)],
                        acc.at[pl.ds(s * _RSPAN + zoff, zsz)])
        zoff += zsz

    @pl.when(s == _NS - 1)
    def _zero_tail():
        pltpu.sync_copy(rows_v.at[pl.ds(0, _TAIL)],
                        acc.at[pl.ds(_NS * _RSPAN, _TAIL)])

    plsc.subcore_barrier()

    base = (c * _NS + s) * _EPW

    def _scale_rows(rows_ref, wts_ref, count):
        def _one_group(g, carry):
            wv = wts_ref[pl.ds(g * 16, 16)]
            for j in range(16):
                e = g * 16 + j
                wb = jnp.full((16,), wv[j], jnp.float32)
                for k in range(_NLANE):
                    rows_ref[e, pl.ds(k * 16, 16)] = (
                        rows_ref[e, pl.ds(k * 16, 16)] * wb)
            return carry
        lax.fori_loop(0, count // 16, _one_group, 0)

    def _chunk(i, carry):
        off = base + i * _CH
        pltpu.sync_copy(src_hbm.at[pl.ds(off, _CH)], src_v)
        pltpu.sync_copy(dst_hbm.at[pl.ds(off, _CH)], dst_v)
        pltpu.sync_copy(w_hbm.at[pl.ds(off, _CH)], w_v)
        pltpu.async_copy(x_hbm.at[src_v], rows_v, sem).wait()
        _scale_rows(rows_v, w_v, _CH)
        pltpu.sync_copy(rows_v, acc.at[dst_v], add=True)
        return carry

    lax.fori_loop(0, _NFULL, _chunk, 0)

    if _REM:
        off = base + _NFULL * _CH
        pltpu.sync_copy(src_hbm.at[pl.ds(off, _REM)], src_r)
        pltpu.sync_copy(dst_hbm.at[pl.ds(off, _REM)], dst_r)
        pltpu.sync_copy(w_hbm.at[pl.ds(off, _REM)], w_r)
        pltpu.async_copy(x_hbm.at[src_r], rows_r, sem).wait()
        _scale_rows(rows_r, w_r, _REM)
        pltpu.sync_copy(rows_r, acc.at[dst_r], add=True)

    plsc.subcore_barrier()
    pltpu.sync_copy(acc.at[pl.ds(s * _RSPAN, _RSPAN)],
                    out_hbm.at[c, pl.ds(s * _RSPAN, _RSPAN)])

    @pl.when(s == _NS - 1)
    def _publish_tail():
        pltpu.sync_copy(acc.at[pl.ds(_NS * _RSPAN, _TAIL)],
                        out_hbm.at[c, pl.ds(_NS * _RSPAN, _TAIL)])


_BM = 1000


def _tc_body(p_ref, w_ref, o_ref):
    agg = p_ref[0] + p_ref[1]
    o_ref[...] = jnp.maximum(
        jnp.dot(agg, w_ref[...], preferred_element_type=jnp.float32), 0.0)


def _tc_matmul_relu(partials, W):
    return pl.pallas_call(
        _tc_body,
        grid=(_N // _BM,),
        in_specs=[
            pl.BlockSpec((_NC, _BM, _D), lambda i: (0, i, 0)),
            pl.BlockSpec((_D, _D), lambda i: (0, 0)),
        ],
        out_specs=pl.BlockSpec((_BM, _D), lambda i: (i, 0)),
        out_shape=jax.ShapeDtypeStruct((_N, _D), jnp.float32),
    )(partials, W)


@jax.jit
def kernel(x, edge_index, edge_weight, W):
    src = edge_index[0].astype(jnp.int32)
    dst = edge_index[1].astype(jnp.int32)
    partials = _sc_aggregate(x, src, dst, edge_weight.astype(jnp.float32))
    return _tc_matmul_relu(partials, W)
